# Initial kernel scaffold; baseline (speedup 1.0000x reference)
import jax
import jax.numpy as jnp
from jax.experimental import pallas as pl


def kernel(g, x, etypes, norm, weight, h_bias):
    raise NotImplementedError("write your pallas kernel here")


# speedup vs baseline: 10.3614x; 10.3614x over previous
import functools

import jax
import jax.numpy as jnp
from jax import lax
from jax.experimental import pallas as pl
from jax.experimental.pallas import tpu as pltpu
from jax.experimental.pallas import tpu_sc as plsc

L = 16
NC = 2
NS = 16
CHUNK = 80


def _matmul_body(x_ref, w_ref, o_ref):
    o_ref[0] = jnp.dot(x_ref[...], w_ref[0], preferred_element_type=jnp.float32)


def _rel_transforms(x, weight):
    n, din = x.shape
    r, _, dout = weight.shape
    bn = 1000
    return pl.pallas_call(
        _matmul_body,
        grid=(r, n // bn),
        in_specs=[
            pl.BlockSpec((bn, din), lambda ri, ni: (ni, 0)),
            pl.BlockSpec((1, din, dout), lambda ri, ni: (ri, 0, 0)),
        ],
        out_specs=pl.BlockSpec((1, bn, dout), lambda ri, ni: (ri, ni, 0)),
        out_shape=jax.ShapeDtypeStruct((r, n, dout), jnp.float32),
    )(x, weight)


def _combine_body(p_ref, b_ref, o_ref):
    o_ref[...] = p_ref[0] + p_ref[1] + b_ref[...]


def _combine(partial, h_bias):
    nc, n, dout = partial.shape
    bn = 1000
    return pl.pallas_call(
        _combine_body,
        grid=(n // bn,),
        in_specs=[
            pl.BlockSpec((nc, bn, dout), lambda i: (0, i, 0)),
            pl.BlockSpec((1, dout), lambda i: (0, 0)),
        ],
        out_specs=pl.BlockSpec((bn, dout), lambda i: (i, 0)),
        out_shape=jax.ShapeDtypeStruct((n, dout), jnp.float32),
    )(partial, h_bias.reshape(1, dout))


def _make_sc_scatter(n, dout, e):
    nw = NC * NS
    ew = e // nw
    nchunks = ew // CHUNK
    rstride = 624
    rcnt = 640
    zrows = 128
    nzcopies = rcnt // zrows
    nj = dout // L
    mesh = plsc.VectorSubcoreMesh(core_axis_name="c", subcore_axis_name="s")

    @functools.partial(
        pl.kernel,
        out_type=jax.ShapeDtypeStruct((NC, n, dout), jnp.float32),
        mesh=mesh,
        scratch_types=[
            pltpu.VMEM((CHUNK,), jnp.int32),
            pltpu.VMEM((CHUNK,), jnp.int32),
            pltpu.VMEM((CHUNK,), jnp.int32),
            pltpu.VMEM((CHUNK,), jnp.float32),
            pltpu.VMEM((CHUNK,), jnp.int32),
            pltpu.VMEM((CHUNK, 128), jnp.float32),
            pltpu.VMEM((zrows, 128), jnp.float32),
            pltpu.VMEM_SHARED((n, dout), jnp.float32),
            pltpu.SemaphoreType.DMA,
        ],
    )
    def sc_kernel(src_hbm, et_hbm, dst_hbm, norm_hbm, h_hbm, out_hbm,
                  src_v, et_v, dst_v, norm_v, idx_v, rows_v, zero_v, acc, sem):
        c = lax.axis_index("c")
        s = lax.axis_index("s")
        wid = c * NS + s

        def zstore(i, _):
            for j in range(nj):
                zero_v[i, pl.ds(j * L, L)] = jnp.zeros((L,), jnp.float32)
            return 0
        lax.fori_loop(0, zrows, zstore, 0)
        for t in range(nzcopies):
            pltpu.sync_copy(zero_v, acc.at[pl.ds(s * rstride + t * zrows, zrows)])
        plsc.subcore_barrier()

        def chunk_body(k, _):
            base = wid * ew + k * CHUNK
            pltpu.sync_copy(src_hbm.at[pl.ds(base, CHUNK)], src_v)
            pltpu.sync_copy(et_hbm.at[pl.ds(base, CHUNK)], et_v)
            pltpu.sync_copy(dst_hbm.at[pl.ds(base, CHUNK)], dst_v)
            pltpu.sync_copy(norm_hbm.at[pl.ds(base, CHUNK)], norm_v)
            for j in range(CHUNK // L):
                sl = pl.ds(j * L, L)
                idx_v[sl] = et_v[sl] * n + src_v[sl]
            pltpu.async_copy(h_hbm.at[idx_v], rows_v, sem).wait()

            def group_body(gi, _):
                nv = norm_v[pl.ds(gi * L, L)]
                for lane in range(L):
                    ei = gi * L + lane
                    nb = jnp.full((L,), nv[lane], jnp.float32)
                    for j in range(nj):
                        sl = pl.ds(j * L, L)
                        rows_v[ei, sl] = rows_v[ei, sl] * nb
                return 0
            lax.fori_loop(0, CHUNK // L, group_body, 0)

            pltpu.sync_copy(rows_v, acc.at[dst_v], add=True)
            return 0
        lax.fori_loop(0, nchunks, chunk_body, 0)

        plsc.subcore_barrier()
        pltpu.sync_copy(acc.at[pl.ds(s * rstride, rcnt)],
                        out_hbm.at[c, pl.ds(s * rstride, rcnt)])

    return sc_kernel


def kernel(g, x, etypes, norm, weight, h_bias):
    n, din = x.shape
    r, _, dout = weight.shape
    e = g.shape[1]
    h_flat = _rel_transforms(x, weight).reshape(r * n, dout)
    partial = _make_sc_scatter(n, dout, e)(
        g[0], etypes, g[1], norm.reshape(-1), h_flat)
    return _combine(partial, h_bias)


# speedup vs baseline: 20.9163x; 2.0187x over previous
import functools

import jax
import jax.numpy as jnp
from jax import lax
from jax.experimental import pallas as pl
from jax.experimental.pallas import tpu as pltpu
from jax.experimental.pallas import tpu_sc as plsc

L = 16
NC = 2
NS = 16
CHUNK = 80


def _matmul_body(x_ref, w_ref, o_ref):
    o_ref[0] = jnp.dot(x_ref[...], w_ref[0], preferred_element_type=jnp.float32)


def _rel_transforms(x, weight):
    n, din = x.shape
    r, _, dout = weight.shape
    bn = 1000
    return pl.pallas_call(
        _matmul_body,
        grid=(r, n // bn),
        in_specs=[
            pl.BlockSpec((bn, din), lambda ri, ni: (ni, 0)),
            pl.BlockSpec((1, din, dout), lambda ri, ni: (ri, 0, 0)),
        ],
        out_specs=pl.BlockSpec((1, bn, dout), lambda ri, ni: (ri, ni, 0)),
        out_shape=jax.ShapeDtypeStruct((r, n, dout), jnp.float32),
    )(x, weight)


def _combine_body(p_ref, b_ref, o_ref):
    o_ref[...] = p_ref[0] + p_ref[1] + b_ref[...]


def _combine(partial, h_bias):
    nc, n, dout = partial.shape
    bn = 1000
    return pl.pallas_call(
        _combine_body,
        grid=(n // bn,),
        in_specs=[
            pl.BlockSpec((nc, bn, dout), lambda i: (0, i, 0)),
            pl.BlockSpec((1, dout), lambda i: (0, 0)),
        ],
        out_specs=pl.BlockSpec((bn, dout), lambda i: (i, 0)),
        out_shape=jax.ShapeDtypeStruct((n, dout), jnp.float32),
    )(partial, h_bias.reshape(1, dout))


def _make_sc_scatter(n, dout, e):
    nw = NC * NS
    ew = e // nw
    nchunks = ew // CHUNK
    rstride = 624
    rcnt = 640
    zrows = 128
    nzcopies = rcnt // zrows
    nj = dout // L
    mesh = plsc.VectorSubcoreMesh(core_axis_name="c", subcore_axis_name="s")

    mw = 3 * CHUNK
    gmax = nw * nchunks - 1

    @functools.partial(
        pl.kernel,
        out_type=jax.ShapeDtypeStruct((NC, n, dout), jnp.float32),
        mesh=mesh,
        scratch_types=[
            pltpu.VMEM((mw,), jnp.int32),
            pltpu.VMEM((mw,), jnp.int32),
            pltpu.VMEM((CHUNK,), jnp.int32),
            pltpu.VMEM((CHUNK,), jnp.int32),
            pltpu.VMEM((CHUNK,), jnp.int32),
            pltpu.VMEM((CHUNK,), jnp.float32),
            pltpu.VMEM((CHUNK,), jnp.float32),
            pltpu.VMEM((CHUNK, 128), jnp.float32),
            pltpu.VMEM((CHUNK, 128), jnp.float32),
            pltpu.VMEM_SHARED((n, dout), jnp.float32),
            pltpu.SemaphoreType.DMA,
            pltpu.SemaphoreType.DMA,
            pltpu.SemaphoreType.DMA,
            pltpu.SemaphoreType.DMA,
            pltpu.SemaphoreType.DMA,
            pltpu.SemaphoreType.DMA,
        ],
    )
    def sc_kernel(meta_hbm, norm_hbm, h_hbm, out_hbm,
                  meta_v0, meta_v1, idx_v0, idx_v1, dst_v, norm_v0, norm_v1,
                  rows_v0, rows_v1,
                  acc, semm0, semm1, semn0, semn1, semg0, semg1):
        meta_v = [meta_v0, meta_v1]
        idx_v = [idx_v0, idx_v1]
        norm_v = [norm_v0, norm_v1]
        rows_v = [rows_v0, rows_v1]
        semm = [semm0, semm1]
        semn = [semn0, semn1]
        semg = [semg0, semg1]
        c = lax.axis_index("c")
        s = lax.axis_index("s")
        wid = c * NS + s

        def meta_start(k, b):
            g = jnp.minimum(wid * nchunks + k, gmax)
            pltpu.async_copy(
                meta_hbm.at[pl.ds(g * mw, mw)], meta_v[b], semm[b])
            pltpu.async_copy(
                norm_hbm.at[pl.ds(g * CHUNK, CHUNK)], norm_v[b], semn[b])

        def meta_wait(b):
            pltpu.make_async_copy(
                meta_hbm.at[pl.ds(0, mw)], meta_v[b], semm[b]).wait()
            pltpu.make_async_copy(
                norm_hbm.at[pl.ds(0, CHUNK)], norm_v[b], semn[b]).wait()

        def idx_compute(b):
            for j in range(CHUNK // L):
                sl = pl.ds(j * L, L)
                idx_v[b][sl] = (meta_v[b][pl.ds(CHUNK + j * L, L)] * n
                                + meta_v[b][sl])

        def gather_start(b):
            return pltpu.async_copy(h_hbm.at[idx_v[b]], rows_v[b], semg[b])

        def gather_wait(b):
            pltpu.make_async_copy(
                h_hbm.at[idx_v[b]], rows_v[b], semg[b]).wait()

        def process_compute(b):
            for j in range(CHUNK // L):
                sl = pl.ds(j * L, L)
                dst_v[sl] = meta_v[b][pl.ds(2 * CHUNK + j * L, L)]

            def group_body(gi, _):
                nv = norm_v[b][pl.ds(gi * L, L)]
                for lane in range(L):
                    ei = gi * L + lane
                    nb = jnp.full((L,), nv[lane], jnp.float32)
                    for j in range(nj):
                        sl = pl.ds(j * L, L)
                        rows_v[b][ei, sl] = rows_v[b][ei, sl] * nb
                return 0
            lax.fori_loop(0, CHUNK // L, group_body, 0)

        def scatter(b):
            pltpu.sync_copy(rows_v[b], acc.at[dst_v], add=True)

        def zstore(i, _):
            for j in range(nj):
                rows_v0[i, pl.ds(j * L, L)] = jnp.zeros((L,), jnp.float32)
            return 0
        lax.fori_loop(0, CHUNK, zstore, 0)
        for t in range(rcnt // CHUNK):
            pltpu.sync_copy(rows_v0,
                            acc.at[pl.ds(s * rstride + t * CHUNK, CHUNK)])
        plsc.subcore_barrier()

        meta_start(0, 0)
        meta_wait(0)
        idx_compute(0)
        gather_start(0)
        meta_start(1, 1)

        def half_step(k, cur, nxt):
            meta_wait(nxt)
            idx_compute(nxt)
            gather_wait(cur)
            gather_start(nxt)
            process_compute(cur)
            meta_start(k + 2, cur)
            scatter(cur)

        def pair_body(t, _):
            half_step(2 * t, 0, 1)
            half_step(2 * t + 1, 1, 0)
            return 0
        lax.fori_loop(0, (nchunks - 1) // 2, pair_body, 0)

        meta_wait(1)
        gather_wait(0)
        process_compute(0)
        scatter(0)

        plsc.subcore_barrier()
        pltpu.sync_copy(acc.at[pl.ds(s * rstride, rcnt)],
                        out_hbm.at[c, pl.ds(s * rstride, rcnt)])

    return sc_kernel


def kernel(g, x, etypes, norm, weight, h_bias):
    n, din = x.shape
    r, _, dout = weight.shape
    e = g.shape[1]
    nch = e // CHUNK
    meta = jnp.concatenate(
        [g[0].reshape(nch, CHUNK), etypes.reshape(nch, CHUNK),
         g[1].reshape(nch, CHUNK)],
        axis=1).reshape(-1)
    h_flat = _rel_transforms(x, weight).reshape(r * n, dout)
    partial = _make_sc_scatter(n, dout, e)(meta, norm.reshape(-1), h_flat)
    return _combine(partial, h_bias)
